# Initial kernel scaffold; baseline (speedup 1.0000x reference)
import jax
import jax.numpy as jnp
from jax.experimental import pallas as pl


def kernel(x, edge_index, W1_l, b1, W1_r, W2_l, b2, W2_r):
    raise NotImplementedError("write your pallas kernel here")


# speedup vs baseline: 8.7497x; 8.7497x over previous
import jax
import jax.numpy as jnp
from jax import lax
from jax.experimental import pallas as pl
from jax.experimental.pallas import tpu as pltpu
from jax.experimental.pallas import tpu_sc as plsc

N = 10000
E = 320000
D_IN = 128
D_HID = 64
D_OUT = 128

NC = 2
NS = 16
NW = NC * NS
EPW = E // NW
CH = 80
NCHUNK = EPW // CH
NP = 10240
RPT = NP // NS
DW = 16


def _make_agg(with_deg):
    mesh = plsc.VectorSubcoreMesh(core_axis_name="c", subcore_axis_name="s")
    outs = [jax.ShapeDtypeStruct((NC, NP, D_HID), jnp.float32)]
    scratch = [
        pltpu.VMEM((NCHUNK, CH), jnp.int32),
        pltpu.VMEM((NCHUNK, CH), jnp.int32),
        pltpu.VMEM((CH, D_HID), jnp.float32),
        pltpu.VMEM_SHARED((NP, D_HID), jnp.float32),
        pltpu.SemaphoreType.DMA,
    ]
    if with_deg:
        outs.append(jax.ShapeDtypeStruct((NC, NP, DW), jnp.float32))
        scratch += [
            pltpu.VMEM((CH, DW), jnp.float32),
            pltpu.VMEM_SHARED((NP, DW), jnp.float32),
        ]

    def body(*refs):
        if with_deg:
            (table, srcs, dsts, zer_d, zer_w, one_w,
             out_acc, out_deg,
             src_v, dst_v, buf, acc, sem, ones_v, dacc) = refs
        else:
            (table, srcs, dsts, zer_d,
             out_acc,
             src_v, dst_v, buf, acc, sem) = refs
        c = lax.axis_index("c")
        s = lax.axis_index("s")
        wid = s * NC + c

        pltpu.sync_copy(srcs.at[wid], src_v)
        pltpu.sync_copy(dsts.at[wid], dst_v)
        pltpu.sync_copy(zer_d.at[pl.ds(s * RPT, RPT)],
                        acc.at[pl.ds(s * RPT, RPT)])
        if with_deg:
            pltpu.sync_copy(one_w, ones_v)
            pltpu.sync_copy(zer_w.at[pl.ds(s * RPT, RPT)],
                            dacc.at[pl.ds(s * RPT, RPT)])
        plsc.subcore_barrier()

        def step(j, carry):
            pltpu.async_copy(table.at[src_v.at[j]], buf, sem).wait()
            pltpu.sync_copy(buf, acc.at[dst_v.at[j]], add=True)
            if with_deg:
                pltpu.sync_copy(ones_v, dacc.at[dst_v.at[j]], add=True)
            return carry

        lax.fori_loop(0, NCHUNK, step, 0)
        plsc.subcore_barrier()

        pltpu.sync_copy(acc.at[pl.ds(s * RPT, RPT)],
                        out_acc.at[c, pl.ds(s * RPT, RPT)])
        if with_deg:
            pltpu.sync_copy(dacc.at[pl.ds(s * RPT, RPT)],
                            out_deg.at[c, pl.ds(s * RPT, RPT)])

    return pl.kernel(body, out_type=tuple(outs), mesh=mesh,
                     scratch_types=scratch,
                     compiler_params=pltpu.CompilerParams(
                         use_tc_tiling_on_sc=False))


_agg_deg = _make_agg(with_deg=True)
_agg = _make_agg(with_deg=False)


_BN = 2000


def _mm_body(x_ref, w_ref, o_ref):
    o_ref[...] = jnp.dot(x_ref[...], w_ref[...],
                         preferred_element_type=jnp.float32)


def _tc_project(x, w):
    return pl.pallas_call(
        _mm_body,
        grid=(N // _BN,),
        in_specs=[pl.BlockSpec((_BN, D_IN), lambda i: (i, 0)),
                  pl.BlockSpec((D_IN, D_IN), lambda i: (0, 0))],
        out_specs=pl.BlockSpec((_BN, D_IN), lambda i: (i, 0)),
        out_shape=jax.ShapeDtypeStruct((N, D_IN), jnp.float32),
    )(x, w)


def _post1_body(a_ref, d_ref, r_ref, b_ref, o_ref):
    a = a_ref[0] + a_ref[1]
    deg = jnp.maximum(d_ref[0, :, 0] + d_ref[1, :, 0], 1.0)
    o_ref[...] = jnp.maximum(a / deg[:, None] + r_ref[...] + b_ref[...], 0.0)


def _tc_post1(agg, degp, r1, b1):
    return pl.pallas_call(
        _post1_body,
        grid=(N // _BN,),
        in_specs=[pl.BlockSpec((NC, _BN, D_HID), lambda i: (0, i, 0)),
                  pl.BlockSpec((NC, _BN, DW), lambda i: (0, i, 0)),
                  pl.BlockSpec((_BN, D_HID), lambda i: (i, 0)),
                  pl.BlockSpec((1, D_HID), lambda i: (0, 0))],
        out_specs=pl.BlockSpec((_BN, D_HID), lambda i: (i, 0)),
        out_shape=jax.ShapeDtypeStruct((N, D_HID), jnp.float32),
    )(agg, degp, r1, b1)


def _post2_body(a_ref, d_ref, h_ref, w_ref, b_ref, o_ref):
    a = a_ref[0] + a_ref[1]
    deg = jnp.maximum(d_ref[0, :, 0] + d_ref[1, :, 0], 1.0)
    mean = a / deg[:, None]
    cat = jnp.concatenate([mean, h_ref[...]], axis=1)
    o = jnp.dot(cat, w_ref[...], preferred_element_type=jnp.float32)
    o = o + b_ref[...]
    m = jnp.max(o, axis=1, keepdims=True)
    ls = jnp.log(jnp.sum(jnp.exp(o - m), axis=1, keepdims=True))
    o_ref[...] = o - m - ls


def _tc_post2(agg, degp, h, w, b2):
    return pl.pallas_call(
        _post2_body,
        grid=(N // _BN,),
        in_specs=[pl.BlockSpec((NC, _BN, D_HID), lambda i: (0, i, 0)),
                  pl.BlockSpec((NC, _BN, DW), lambda i: (0, i, 0)),
                  pl.BlockSpec((_BN, D_HID), lambda i: (i, 0)),
                  pl.BlockSpec((D_IN, D_OUT), lambda i: (0, 0)),
                  pl.BlockSpec((1, D_OUT), lambda i: (0, 0))],
        out_specs=pl.BlockSpec((_BN, D_OUT), lambda i: (i, 0)),
        out_shape=jax.ShapeDtypeStruct((N, D_OUT), jnp.float32),
    )(agg, degp, h, w, b2)


def kernel(x, edge_index, W1_l, b1, W1_r, W2_l, b2, W2_r):
    src = edge_index[0].reshape(NW, NCHUNK, CH)
    dst = edge_index[1].reshape(NW, NCHUNK, CH)

    z = _tc_project(x, jnp.concatenate([W1_l, W1_r], axis=1))
    y1 = z[:, :D_HID]
    r1 = z[:, D_HID:]

    zer_d = jnp.zeros((NP, D_HID), jnp.float32)
    zer_w = jnp.zeros((NP, DW), jnp.float32)
    one_w = jnp.ones((CH, DW), jnp.float32)

    agg1, degp = _agg_deg(y1, src, dst, zer_d, zer_w, one_w)
    agg1, degp = agg1[:, :N], degp[:, :N]
    h = _tc_post1(agg1, degp, r1, b1.reshape(1, D_HID))

    agg2, = _agg(h, src, dst, zer_d)
    agg2 = agg2[:, :N]
    return _tc_post2(agg2, degp, h,
                     jnp.concatenate([W2_l, W2_r], axis=0),
                     b2.reshape(1, D_OUT))


# speedup vs baseline: 13.8897x; 1.5875x over previous
import jax
import jax.numpy as jnp
from jax import lax
from jax.experimental import pallas as pl
from jax.experimental.pallas import tpu as pltpu
from jax.experimental.pallas import tpu_sc as plsc

N = 10000
E = 320000
D_IN = 128
D_HID = 64
D_OUT = 128

NC = 2
NS = 16
NW = NC * NS
EPW = E // NW
CH = 125
NCHUNK = EPW // CH
NP = 10240
RPT = NP // NS
DW = 16


def _make_agg(with_deg):
    mesh = plsc.VectorSubcoreMesh(core_axis_name="c", subcore_axis_name="s")
    outs = [jax.ShapeDtypeStruct((NC, NP, D_HID), jnp.float32)]
    scratch = [
        pltpu.VMEM((NCHUNK, CH), jnp.int32),
        pltpu.VMEM((NCHUNK, CH), jnp.int32),
        pltpu.VMEM((CH, D_HID), jnp.float32),
        pltpu.VMEM((CH, D_HID), jnp.float32),
        pltpu.VMEM_SHARED((NP, D_HID), jnp.float32),
        pltpu.SemaphoreType.DMA,
        pltpu.SemaphoreType.DMA,
    ]
    if with_deg:
        outs.append(jax.ShapeDtypeStruct((NC, NP, DW), jnp.float32))
        scratch += [
            pltpu.VMEM((CH, DW), jnp.float32),
            pltpu.VMEM_SHARED((NP, DW), jnp.float32),
        ]

    def body(*refs):
        if with_deg:
            (table, srcs, dsts, zer_d, zer_w, one_w,
             out_acc, out_deg,
             src_v, dst_v, buf_a, buf_b, acc, sem_a, sem_b,
             ones_v, dacc) = refs
        else:
            (table, srcs, dsts, zer_d,
             out_acc,
             src_v, dst_v, buf_a, buf_b, acc, sem_a, sem_b) = refs
        c = lax.axis_index("c")
        s = lax.axis_index("s")
        wid = s * NC + c

        pltpu.sync_copy(srcs.at[wid], src_v)
        pltpu.sync_copy(dsts.at[wid], dst_v)
        pltpu.sync_copy(zer_d.at[pl.ds(s * RPT, RPT)],
                        acc.at[pl.ds(s * RPT, RPT)])
        if with_deg:
            pltpu.sync_copy(one_w, ones_v)
            pltpu.sync_copy(zer_w.at[pl.ds(s * RPT, RPT)],
                            dacc.at[pl.ds(s * RPT, RPT)])
        plsc.subcore_barrier()

        pltpu.async_copy(table.at[src_v.at[0]], buf_a, sem_a)
        pltpu.async_copy(table.at[src_v.at[1]], buf_b, sem_b)

        def pair(i, carry):
            j = 2 * i
            pltpu.make_async_copy(table.at[src_v.at[j]], buf_a, sem_a).wait()
            pltpu.sync_copy(buf_a, acc.at[dst_v.at[j]], add=True)
            if with_deg:
                pltpu.sync_copy(ones_v, dacc.at[dst_v.at[j]], add=True)

            @pl.when(j + 2 < NCHUNK)
            def _():
                pltpu.async_copy(table.at[src_v.at[j + 2]], buf_a, sem_a)

            pltpu.make_async_copy(table.at[src_v.at[j + 1]], buf_b,
                                  sem_b).wait()
            pltpu.sync_copy(buf_b, acc.at[dst_v.at[j + 1]], add=True)
            if with_deg:
                pltpu.sync_copy(ones_v, dacc.at[dst_v.at[j + 1]], add=True)

            @pl.when(j + 3 < NCHUNK)
            def _():
                pltpu.async_copy(table.at[src_v.at[j + 3]], buf_b, sem_b)

            return carry

        lax.fori_loop(0, NCHUNK // 2, pair, 0)
        plsc.subcore_barrier()

        pltpu.sync_copy(acc.at[pl.ds(s * RPT, RPT)],
                        out_acc.at[c, pl.ds(s * RPT, RPT)])
        if with_deg:
            pltpu.sync_copy(dacc.at[pl.ds(s * RPT, RPT)],
                            out_deg.at[c, pl.ds(s * RPT, RPT)])

    return pl.kernel(body, out_type=tuple(outs), mesh=mesh,
                     scratch_types=scratch,
                     compiler_params=pltpu.CompilerParams(
                         use_tc_tiling_on_sc=False))


_agg_deg = _make_agg(with_deg=True)
_agg = _make_agg(with_deg=False)


_BN = 2000


def _mm_body(x_ref, w_ref, o_ref):
    o_ref[...] = jnp.dot(x_ref[...], w_ref[...],
                         preferred_element_type=jnp.float32)


def _tc_project(x, w):
    return pl.pallas_call(
        _mm_body,
        grid=(N // _BN,),
        in_specs=[pl.BlockSpec((_BN, D_IN), lambda i: (i, 0)),
                  pl.BlockSpec((D_IN, D_IN), lambda i: (0, 0))],
        out_specs=pl.BlockSpec((_BN, D_IN), lambda i: (i, 0)),
        out_shape=jax.ShapeDtypeStruct((N, D_IN), jnp.float32),
    )(x, w)


def _post1_body(a_ref, d_ref, r_ref, b_ref, o_ref):
    a = a_ref[0] + a_ref[1]
    deg = jnp.maximum(d_ref[0, :, 0] + d_ref[1, :, 0], 1.0)
    o_ref[...] = jnp.maximum(a / deg[:, None] + r_ref[...] + b_ref[...], 0.0)


def _tc_post1(agg, degp, r1, b1):
    return pl.pallas_call(
        _post1_body,
        grid=(N // _BN,),
        in_specs=[pl.BlockSpec((NC, _BN, D_HID), lambda i: (0, i, 0)),
                  pl.BlockSpec((NC, _BN, DW), lambda i: (0, i, 0)),
                  pl.BlockSpec((_BN, D_HID), lambda i: (i, 0)),
                  pl.BlockSpec((1, D_HID), lambda i: (0, 0))],
        out_specs=pl.BlockSpec((_BN, D_HID), lambda i: (i, 0)),
        out_shape=jax.ShapeDtypeStruct((N, D_HID), jnp.float32),
    )(agg, degp, r1, b1)


def _post2_body(a_ref, d_ref, h_ref, w_ref, b_ref, o_ref):
    a = a_ref[0] + a_ref[1]
    deg = jnp.maximum(d_ref[0, :, 0] + d_ref[1, :, 0], 1.0)
    mean = a / deg[:, None]
    cat = jnp.concatenate([mean, h_ref[...]], axis=1)
    o = jnp.dot(cat, w_ref[...], preferred_element_type=jnp.float32)
    o = o + b_ref[...]
    m = jnp.max(o, axis=1, keepdims=True)
    ls = jnp.log(jnp.sum(jnp.exp(o - m), axis=1, keepdims=True))
    o_ref[...] = o - m - ls


def _tc_post2(agg, degp, h, w, b2):
    return pl.pallas_call(
        _post2_body,
        grid=(N // _BN,),
        in_specs=[pl.BlockSpec((NC, _BN, D_HID), lambda i: (0, i, 0)),
                  pl.BlockSpec((NC, _BN, DW), lambda i: (0, i, 0)),
                  pl.BlockSpec((_BN, D_HID), lambda i: (i, 0)),
                  pl.BlockSpec((D_IN, D_OUT), lambda i: (0, 0)),
                  pl.BlockSpec((1, D_OUT), lambda i: (0, 0))],
        out_specs=pl.BlockSpec((_BN, D_OUT), lambda i: (i, 0)),
        out_shape=jax.ShapeDtypeStruct((N, D_OUT), jnp.float32),
    )(agg, degp, h, w, b2)


def kernel(x, edge_index, W1_l, b1, W1_r, W2_l, b2, W2_r):
    src = edge_index[0].reshape(NW, NCHUNK, CH)
    dst = edge_index[1].reshape(NW, NCHUNK, CH)

    z = _tc_project(x, jnp.concatenate([W1_l, W1_r], axis=1))
    y1 = z[:, :D_HID]
    r1 = z[:, D_HID:]

    zer_d = jnp.zeros((NP, D_HID), jnp.float32)
    zer_w = jnp.zeros((NP, DW), jnp.float32)
    one_w = jnp.ones((CH, DW), jnp.float32)

    agg1, degp = _agg_deg(y1, src, dst, zer_d, zer_w, one_w)
    agg1, degp = agg1[:, :N], degp[:, :N]
    h = _tc_post1(agg1, degp, r1, b1.reshape(1, D_HID))

    agg2, = _agg(h, src, dst, zer_d)
    agg2 = agg2[:, :N]
    return _tc_post2(agg2, degp, h,
                     jnp.concatenate([W2_l, W2_r], axis=0),
                     b2.reshape(1, D_OUT))


# speedup vs baseline: 15.5851x; 1.1221x over previous
import jax
import jax.numpy as jnp
from jax import lax
from jax.experimental import pallas as pl
from jax.experimental.pallas import tpu as pltpu
from jax.experimental.pallas import tpu_sc as plsc

N = 10000
E = 320000
D_IN = 128
D_HID = 64
D_OUT = 128

NC = 2
NS = 16
NW = NC * NS
EPW = E // NW
CH = 125
NCHUNK = EPW // CH
NP = 10240
RPT = NP // NS
DW = 16


def _make_agg(with_deg):
    mesh = plsc.VectorSubcoreMesh(core_axis_name="c", subcore_axis_name="s")
    outs = [jax.ShapeDtypeStruct((NC, NP, D_HID), jnp.float32)]
    scratch = [
        pltpu.VMEM((NCHUNK, CH), jnp.int32),
        pltpu.VMEM((NCHUNK, CH), jnp.int32),
        pltpu.VMEM((CH, D_HID), jnp.float32),
        pltpu.VMEM((CH, D_HID), jnp.float32),
        pltpu.VMEM_SHARED((NP, D_HID), jnp.float32),
        pltpu.SemaphoreType.DMA,
        pltpu.SemaphoreType.DMA,
    ]
    if with_deg:
        outs.append(jax.ShapeDtypeStruct((NC, NP, DW), jnp.float32))
        scratch += [
            pltpu.VMEM((CH, DW), jnp.float32),
            pltpu.VMEM_SHARED((NP, DW), jnp.float32),
        ]

    def body(*refs):
        if with_deg:
            (table, edges, zer_d, zer_w, one_w,
             out_acc, out_deg,
             src_v, dst_v, buf_a, buf_b, acc, sem_a, sem_b,
             ones_v, dacc) = refs
        else:
            (table, edges, zer_d,
             out_acc,
             src_v, dst_v, buf_a, buf_b, acc, sem_a, sem_b) = refs
        c = lax.axis_index("c")
        s = lax.axis_index("s")
        wid = s * NC + c

        pltpu.sync_copy(edges.at[0, wid], src_v)
        pltpu.sync_copy(edges.at[1, wid], dst_v)
        pltpu.sync_copy(zer_d, acc.at[pl.ds(s * RPT, RPT)])
        if with_deg:
            pltpu.sync_copy(one_w, ones_v)
            pltpu.sync_copy(zer_w, dacc.at[pl.ds(s * RPT, RPT)])
        plsc.subcore_barrier()

        pltpu.async_copy(table.at[src_v.at[0]], buf_a, sem_a)
        pltpu.async_copy(table.at[src_v.at[1]], buf_b, sem_b)

        def pair(i, carry):
            j = 2 * i
            pltpu.make_async_copy(table.at[src_v.at[j]], buf_a, sem_a).wait()
            pltpu.sync_copy(buf_a, acc.at[dst_v.at[j]], add=True)
            if with_deg:
                pltpu.sync_copy(ones_v, dacc.at[dst_v.at[j]], add=True)

            @pl.when(j + 2 < NCHUNK)
            def _():
                pltpu.async_copy(table.at[src_v.at[j + 2]], buf_a, sem_a)

            pltpu.make_async_copy(table.at[src_v.at[j + 1]], buf_b,
                                  sem_b).wait()
            pltpu.sync_copy(buf_b, acc.at[dst_v.at[j + 1]], add=True)
            if with_deg:
                pltpu.sync_copy(ones_v, dacc.at[dst_v.at[j + 1]], add=True)

            @pl.when(j + 3 < NCHUNK)
            def _():
                pltpu.async_copy(table.at[src_v.at[j + 3]], buf_b, sem_b)

            return carry

        lax.fori_loop(0, NCHUNK // 2, pair, 0)
        plsc.subcore_barrier()

        pltpu.sync_copy(acc.at[pl.ds(s * RPT, RPT)],
                        out_acc.at[c, pl.ds(s * RPT, RPT)])
        if with_deg:
            pltpu.sync_copy(dacc.at[pl.ds(s * RPT, RPT)],
                            out_deg.at[c, pl.ds(s * RPT, RPT)])

    return pl.kernel(body, out_type=tuple(outs), mesh=mesh,
                     scratch_types=scratch,
                     compiler_params=pltpu.CompilerParams(
                         use_tc_tiling_on_sc=False))


_agg_deg = _make_agg(with_deg=True)
_agg = _make_agg(with_deg=False)


_BN = 2000


def _proj_body(x_ref, w_ref, y_ref, r_ref):
    z = jnp.dot(x_ref[...], w_ref[...], preferred_element_type=jnp.float32)
    y_ref[...] = z[:, :D_HID]
    r_ref[...] = z[:, D_HID:]


def _tc_project(x, w):
    return pl.pallas_call(
        _proj_body,
        grid=(N // _BN,),
        in_specs=[pl.BlockSpec((_BN, D_IN), lambda i: (i, 0)),
                  pl.BlockSpec((D_IN, D_IN), lambda i: (0, 0))],
        out_specs=[pl.BlockSpec((_BN, D_HID), lambda i: (i, 0)),
                   pl.BlockSpec((_BN, D_HID), lambda i: (i, 0))],
        out_shape=[jax.ShapeDtypeStruct((N, D_HID), jnp.float32),
                   jax.ShapeDtypeStruct((N, D_HID), jnp.float32)],
    )(x, w)


def _post1_body(a_ref, d_ref, r_ref, b_ref, o_ref):
    a = a_ref[0] + a_ref[1]
    deg = jnp.maximum(d_ref[0, :, 0] + d_ref[1, :, 0], 1.0)
    o_ref[...] = jnp.maximum(a / deg[:, None] + r_ref[...] + b_ref[...], 0.0)


def _tc_post1(agg, degp, r1, b1):
    return pl.pallas_call(
        _post1_body,
        grid=(N // _BN,),
        in_specs=[pl.BlockSpec((NC, _BN, D_HID), lambda i: (0, i, 0)),
                  pl.BlockSpec((NC, _BN, DW), lambda i: (0, i, 0)),
                  pl.BlockSpec((_BN, D_HID), lambda i: (i, 0)),
                  pl.BlockSpec((1, D_HID), lambda i: (0, 0))],
        out_specs=pl.BlockSpec((_BN, D_HID), lambda i: (i, 0)),
        out_shape=jax.ShapeDtypeStruct((N, D_HID), jnp.float32),
    )(agg, degp, r1, b1)


def _post2_body(a_ref, d_ref, h_ref, w_ref, b_ref, o_ref):
    a = a_ref[0] + a_ref[1]
    deg = jnp.maximum(d_ref[0, :, 0] + d_ref[1, :, 0], 1.0)
    mean = a / deg[:, None]
    cat = jnp.concatenate([mean, h_ref[...]], axis=1)
    o = jnp.dot(cat, w_ref[...], preferred_element_type=jnp.float32)
    o = o + b_ref[...]
    m = jnp.max(o, axis=1, keepdims=True)
    ls = jnp.log(jnp.sum(jnp.exp(o - m), axis=1, keepdims=True))
    o_ref[...] = o - m - ls


def _tc_post2(agg, degp, h, w, b2):
    return pl.pallas_call(
        _post2_body,
        grid=(N // _BN,),
        in_specs=[pl.BlockSpec((NC, _BN, D_HID), lambda i: (0, i, 0)),
                  pl.BlockSpec((NC, _BN, DW), lambda i: (0, i, 0)),
                  pl.BlockSpec((_BN, D_HID), lambda i: (i, 0)),
                  pl.BlockSpec((D_IN, D_OUT), lambda i: (0, 0)),
                  pl.BlockSpec((1, D_OUT), lambda i: (0, 0))],
        out_specs=pl.BlockSpec((_BN, D_OUT), lambda i: (i, 0)),
        out_shape=jax.ShapeDtypeStruct((N, D_OUT), jnp.float32),
    )(agg, degp, h, w, b2)


def kernel(x, edge_index, W1_l, b1, W1_r, W2_l, b2, W2_r):
    edges = edge_index.reshape(2, NW, NCHUNK, CH)

    y1, r1 = _tc_project(x, jnp.concatenate([W1_l, W1_r], axis=1))

    zer_d = jnp.zeros((RPT, D_HID), jnp.float32)
    zer_w = jnp.zeros((RPT, DW), jnp.float32)
    one_w = jnp.ones((CH, DW), jnp.float32)

    agg1, degp = _agg_deg(y1, edges, zer_d, zer_w, one_w)
    h = _tc_post1(agg1, degp, r1, b1.reshape(1, D_HID))

    agg2, = _agg(h, edges, zer_d)
    return _tc_post2(agg2, degp, h,
                     jnp.concatenate([W2_l, W2_r], axis=0),
                     b2.reshape(1, D_OUT))


# speedup vs baseline: 17.9656x; 1.1527x over previous
import jax
import jax.numpy as jnp
from jax import lax
from jax.experimental import pallas as pl
from jax.experimental.pallas import tpu as pltpu
from jax.experimental.pallas import tpu_sc as plsc

N = 10000
E = 320000
D_IN = 128
D_HID = 64
D_OUT = 128

NC = 2
NS = 16
NW = NC * NS
EPW = E // NW
CH = 125
NCHUNK = EPW // CH
NP = 10240
RPT = NP // NS
DW = 16


def _make_agg(with_deg):
    mesh = plsc.VectorSubcoreMesh(core_axis_name="c", subcore_axis_name="s")
    outs = [jax.ShapeDtypeStruct((NC, NP, D_HID), jnp.float32)]
    scratch = [
        pltpu.VMEM((NCHUNK, CH), jnp.int32),
        pltpu.VMEM((NCHUNK, CH), jnp.int32),
        pltpu.VMEM((CH, D_HID), jnp.float32),
        pltpu.VMEM((CH, D_HID), jnp.float32),
        pltpu.VMEM((CH, D_HID), jnp.float32),
        pltpu.VMEM((CH, D_HID), jnp.float32),
        pltpu.VMEM_SHARED((NP, D_HID), jnp.float32),
        pltpu.SemaphoreType.DMA,
        pltpu.SemaphoreType.DMA,
        pltpu.SemaphoreType.DMA,
        pltpu.SemaphoreType.DMA,
    ]
    if with_deg:
        outs.append(jax.ShapeDtypeStruct((NC, NP, DW), jnp.float32))
        scratch += [
            pltpu.VMEM((CH, DW), jnp.float32),
            pltpu.VMEM_SHARED((NP, DW), jnp.float32),
            pltpu.SemaphoreType.DMA,
        ]

    def body(*refs):
        if with_deg:
            (table, edges, zer_d, zer_w, one_w,
             out_acc, out_deg,
             src_v, dst_v, b0, b1_, b2_, b3, acc, s0, s1, s2, s3,
             ones_v, dacc, dsem) = refs
        else:
            (table, edges, zer_d,
             out_acc,
             src_v, dst_v, b0, b1_, b2_, b3, acc, s0, s1, s2, s3) = refs
        bufs = (b0, b1_, b2_, b3)
        sems = (s0, s1, s2, s3)
        c = lax.axis_index("c")
        s = lax.axis_index("s")
        wid = s * NC + c

        pltpu.sync_copy(edges.at[0, wid], src_v)
        pltpu.sync_copy(edges.at[1, wid], dst_v)
        pltpu.sync_copy(zer_d, acc.at[pl.ds(s * RPT, RPT)])
        if with_deg:
            pltpu.sync_copy(one_w, ones_v)
            pltpu.sync_copy(zer_w, dacc.at[pl.ds(s * RPT, RPT)])
        plsc.subcore_barrier()

        for k in range(4):
            pltpu.async_copy(table.at[src_v.at[k]], bufs[k], sems[k])

        def group(i, carry):
            j = 4 * i
            for k in range(4):
                pltpu.make_async_copy(table.at[src_v.at[j + k]], bufs[k],
                                      sems[k]).wait()
                pltpu.sync_copy(bufs[k], acc.at[dst_v.at[j + k]], add=True)
                if with_deg:
                    pltpu.async_copy(ones_v, dacc.at[dst_v.at[j + k]], dsem)

                @pl.when(j + k + 4 < NCHUNK)
                def _():
                    pltpu.async_copy(table.at[src_v.at[j + k + 4]], bufs[k],
                                     sems[k])

            return carry

        lax.fori_loop(0, NCHUNK // 4, group, 0)
        if with_deg:
            def drain(i, carry):
                pltpu.make_async_copy(ones_v, dacc.at[dst_v.at[0]],
                                      dsem).wait()
                return carry
            lax.fori_loop(0, NCHUNK, drain, 0)
        plsc.subcore_barrier()

        pltpu.sync_copy(acc.at[pl.ds(s * RPT, RPT)],
                        out_acc.at[c, pl.ds(s * RPT, RPT)])
        if with_deg:
            pltpu.sync_copy(dacc.at[pl.ds(s * RPT, RPT)],
                            out_deg.at[c, pl.ds(s * RPT, RPT)])

    return pl.kernel(body, out_type=tuple(outs), mesh=mesh,
                     scratch_types=scratch,
                     compiler_params=pltpu.CompilerParams(
                         use_tc_tiling_on_sc=False))


_agg_deg = _make_agg(with_deg=True)
_agg = _make_agg(with_deg=False)


_BN = 2000


def _proj_body(x_ref, w_ref, y_ref, r_ref):
    z = jnp.dot(x_ref[...], w_ref[...], preferred_element_type=jnp.float32)
    y_ref[...] = z[:, :D_HID]
    r_ref[...] = z[:, D_HID:]


def _tc_project(x, w):
    return pl.pallas_call(
        _proj_body,
        grid=(N // _BN,),
        in_specs=[pl.BlockSpec((_BN, D_IN), lambda i: (i, 0)),
                  pl.BlockSpec((D_IN, D_IN), lambda i: (0, 0))],
        out_specs=[pl.BlockSpec((_BN, D_HID), lambda i: (i, 0)),
                   pl.BlockSpec((_BN, D_HID), lambda i: (i, 0))],
        out_shape=[jax.ShapeDtypeStruct((N, D_HID), jnp.float32),
                   jax.ShapeDtypeStruct((N, D_HID), jnp.float32)],
    )(x, w)


def _post1_body(a_ref, d_ref, r_ref, b_ref, o_ref):
    a = a_ref[0] + a_ref[1]
    deg = jnp.maximum(d_ref[0, :, 0] + d_ref[1, :, 0], 1.0)
    o_ref[...] = jnp.maximum(a / deg[:, None] + r_ref[...] + b_ref[...], 0.0)


def _tc_post1(agg, degp, r1, b1):
    return pl.pallas_call(
        _post1_body,
        grid=(N // _BN,),
        in_specs=[pl.BlockSpec((NC, _BN, D_HID), lambda i: (0, i, 0)),
                  pl.BlockSpec((NC, _BN, DW), lambda i: (0, i, 0)),
                  pl.BlockSpec((_BN, D_HID), lambda i: (i, 0)),
                  pl.BlockSpec((1, D_HID), lambda i: (0, 0))],
        out_specs=pl.BlockSpec((_BN, D_HID), lambda i: (i, 0)),
        out_shape=jax.ShapeDtypeStruct((N, D_HID), jnp.float32),
    )(agg, degp, r1, b1)


def _post2_body(a_ref, d_ref, h_ref, w_ref, b_ref, o_ref):
    a = a_ref[0] + a_ref[1]
    deg = jnp.maximum(d_ref[0, :, 0] + d_ref[1, :, 0], 1.0)
    mean = a / deg[:, None]
    cat = jnp.concatenate([mean, h_ref[...]], axis=1)
    o = jnp.dot(cat, w_ref[...], preferred_element_type=jnp.float32)
    o = o + b_ref[...]
    m = jnp.max(o, axis=1, keepdims=True)
    ls = jnp.log(jnp.sum(jnp.exp(o - m), axis=1, keepdims=True))
    o_ref[...] = o - m - ls


def _tc_post2(agg, degp, h, w, b2):
    return pl.pallas_call(
        _post2_body,
        grid=(N // _BN,),
        in_specs=[pl.BlockSpec((NC, _BN, D_HID), lambda i: (0, i, 0)),
                  pl.BlockSpec((NC, _BN, DW), lambda i: (0, i, 0)),
                  pl.BlockSpec((_BN, D_HID), lambda i: (i, 0)),
                  pl.BlockSpec((D_IN, D_OUT), lambda i: (0, 0)),
                  pl.BlockSpec((1, D_OUT), lambda i: (0, 0))],
        out_specs=pl.BlockSpec((_BN, D_OUT), lambda i: (i, 0)),
        out_shape=jax.ShapeDtypeStruct((N, D_OUT), jnp.float32),
    )(agg, degp, h, w, b2)


def kernel(x, edge_index, W1_l, b1, W1_r, W2_l, b2, W2_r):
    edges = edge_index.reshape(2, NW, NCHUNK, CH)

    y1, r1 = _tc_project(x, jnp.concatenate([W1_l, W1_r], axis=1))

    zer_d = jnp.zeros((RPT, D_HID), jnp.float32)
    zer_w = jnp.zeros((RPT, DW), jnp.float32)
    one_w = jnp.ones((CH, DW), jnp.float32)

    agg1, degp = _agg_deg(y1, edges, zer_d, zer_w, one_w)
    h = _tc_post1(agg1, degp, r1, b1.reshape(1, D_HID))

    agg2, = _agg(h, edges, zer_d)
    return _tc_post2(agg2, degp, h,
                     jnp.concatenate([W2_l, W2_r], axis=0),
                     b2.reshape(1, D_OUT))
